# Initial kernel scaffold; baseline (speedup 1.0000x reference)
import jax
import jax.numpy as jnp
from jax.experimental import pallas as pl


def kernel(input_ids, unfair_embeds, fair_table, pos_table, token_map):
    raise NotImplementedError("write your pallas kernel here")


# speedup vs baseline: 14.6552x; 14.6552x over previous
import functools

import jax
import jax.numpy as jnp
from jax import lax
from jax.experimental import pallas as pl
from jax.experimental.pallas import tpu as pltpu
from jax.experimental.pallas import tpu_sc as plsc

NC = 2
NS = 16
LANES = 16
NW = NC * NS

CHUNK = 256


def _body(L, n_fair_rows, n_pos_rows,
          ids_hbm, unfair_hbm, fair_hbm, pos_hbm, tm_hbm,
          out_hbm,
          ids_v, fid_v, buf_v, fair_v, pos_v, cnt_v, sem):
    wid = lax.axis_index("s") * NC + lax.axis_index("c")
    n_rows = unfair_hbm.shape[0]
    per_w = n_rows // NW
    n_chunks = per_w // CHUNK

    pltpu.sync_copy(fair_hbm, fair_v)
    pltpu.sync_copy(pos_hbm, pos_v)

    lane_iota = lax.broadcasted_iota(jnp.int32, (LANES,), 0)

    def chunk_body(c, _):
        r0 = wid * per_w + c * CHUNK

        in_cp = pltpu.async_copy(unfair_hbm.at[pl.ds(r0, CHUNK)], buf_v, sem)
        pltpu.sync_copy(ids_hbm.at[pl.ds(r0, CHUNK)], ids_v)

        if True:
            gcps = [
                pltpu.async_copy(tm_hbm.at[ids_v.at[pl.ds(j * 128, 128)]],
                                 fid_v.at[pl.ds(j * 128, 128)], sem)
                for j in range(CHUNK // 128)
            ]
            for cp in gcps:
                cp.wait()
        in_cp.wait()

        def or_tree(x):
            for sh in (1, 2, 4, 8):
                cnt_v[pl.ds(0, LANES)] = x
                x = x | plsc.load_gather(cnt_v,
                                         [(lane_iota + sh) & (LANES - 1)])
            return x

        acc = fid_v[pl.ds(0, LANES)]
        for g in range(1, CHUNK // LANES):
            acc = acc | fid_v[pl.ds(g * LANES, LANES)]

        @pl.when(or_tree(acc)[0] != 0)
        def _dirty_chunk():
            def group_body(g, _):
                fid16 = fid_v[pl.ds(g * LANES, LANES)]
                mask = fid16 != 0

                @pl.when(or_tree(fid16)[0] != 0)
                def _fixup():
                    row16 = g * LANES + lane_iota
                    pos16 = (r0 + row16) % L
                    for c0 in range(buf_v.shape[1]):
                        col = jnp.full((LANES,), c0, jnp.int32)
                        vals = (plsc.load_gather(fair_v, [fid16, col])
                                + plsc.load_gather(pos_v, [pos16, col]))
                        plsc.store_scatter(buf_v, [row16, col], vals,
                                           mask=mask)

                return 0

            lax.fori_loop(0, CHUNK // LANES, group_body, 0)

        pltpu.sync_copy(buf_v, out_hbm.at[pl.ds(r0, CHUNK)])
        return 0

    lax.fori_loop(0, n_chunks, chunk_body, 0)


def kernel(input_ids, unfair_embeds, fair_table, pos_table, token_map):
    B, L = input_ids.shape
    D = unfair_embeds.shape[-1]
    N = B * L
    assert N % (NW * CHUNK) == 0 and CHUNK % 128 == 0 and D == 128

    ids2 = input_ids.reshape(N)
    unfair = unfair_embeds.reshape(N, D)
    pos_sl = pos_table[:L]

    mesh = plsc.VectorSubcoreMesh(core_axis_name="c", subcore_axis_name="s",
                                  num_cores=NC, num_subcores=NS)
    kfn = pl.kernel(
        functools.partial(_body, L, fair_table.shape[0], L),
        out_type=jax.ShapeDtypeStruct((N, D), jnp.float32),
        mesh=mesh,
        scratch_types=[
            pltpu.VMEM((CHUNK,), jnp.int32),
            pltpu.VMEM((CHUNK,), jnp.int32),
            pltpu.VMEM((CHUNK, D), jnp.float32),
            pltpu.VMEM((fair_table.shape[0], D), jnp.float32),
            pltpu.VMEM((L, D), jnp.float32),
            pltpu.VMEM((128,), jnp.int32),
            pltpu.SemaphoreType.DMA,
        ],
        compiler_params=pltpu.CompilerParams(needs_layout_passes=False),
    )
    out = kfn(ids2, unfair, fair_table, pos_sl, token_map)
    return out.reshape(B, L, D)


# speedup vs baseline: 19.0788x; 1.3018x over previous
import functools

import jax
import jax.numpy as jnp
from jax import lax
from jax.experimental import pallas as pl
from jax.experimental.pallas import tpu as pltpu
from jax.experimental.pallas import tpu_sc as plsc

NC = 2
NS = 16
LANES = 16
NW = NC * NS

CHUNK = 320
GSEG = [(o, min(128, CHUNK - o)) for o in range(0, CHUNK, 128)]


def _body(L, ids_hbm, unfair_hbm, fair_hbm, pos_hbm, tm_hbm,
          out_hbm,
          ids_v0, ids_v1, fid_v0, fid_v1, buf_v0, buf_v1,
          fair_v, pos_v, cnt_v,
          sem_in0, sem_in1, sem_out0, sem_out1,
          sem_ids0, sem_ids1, sem_g0, sem_g1):
    ids_v = (ids_v0, ids_v1)
    fid_v = (fid_v0, fid_v1)
    buf_v = (buf_v0, buf_v1)
    sem_in = (sem_in0, sem_in1)
    sem_out = (sem_out0, sem_out1)
    sem_ids = (sem_ids0, sem_ids1)
    sem_g = (sem_g0, sem_g1)

    wid = lax.axis_index("s") * NC + lax.axis_index("c")
    per_w = unfair_hbm.shape[0] // NW
    n_chunks = per_w // CHUNK

    pltpu.sync_copy(fair_hbm, fair_v)
    pltpu.sync_copy(pos_hbm, pos_v)

    lane_iota = lax.broadcasted_iota(jnp.int32, (LANES,), 0)

    def r0_of(c):
        return wid * per_w + c * CHUNK

    def issue_in(c, b):
        pltpu.async_copy(unfair_hbm.at[pl.ds(r0_of(c), CHUNK)],
                         buf_v[b], sem_in[b])

    def wait_in(b):
        pltpu.make_async_copy(unfair_hbm.at[pl.ds(0, CHUNK)],
                              buf_v[b], sem_in[b]).wait()

    def issue_out(c, b):
        pltpu.async_copy(buf_v[b], out_hbm.at[pl.ds(r0_of(c), CHUNK)],
                         sem_out[b])

    def wait_out(b):
        pltpu.make_async_copy(buf_v[b], out_hbm.at[pl.ds(0, CHUNK)],
                              sem_out[b]).wait()

    def issue_ids(c, b):
        pltpu.async_copy(ids_hbm.at[pl.ds(r0_of(c), CHUNK)],
                         ids_v[b], sem_ids[b])

    def wait_ids(b):
        pltpu.make_async_copy(ids_hbm.at[pl.ds(0, CHUNK)],
                              ids_v[b], sem_ids[b]).wait()

    def issue_gather(b):
        for o, w in GSEG:
            pltpu.async_copy(tm_hbm.at[ids_v[b].at[pl.ds(o, w)]],
                             fid_v[b].at[pl.ds(o, w)], sem_g[b])

    def wait_gather(b):
        for o, w in GSEG:
            pltpu.make_async_copy(tm_hbm.at[ids_v[b].at[pl.ds(o, w)]],
                                  fid_v[b].at[pl.ds(o, w)], sem_g[b]).wait()

    def or_tree(x):
        for sh in (1, 2, 4, 8):
            cnt_v[pl.ds(0, LANES)] = x
            x = x | plsc.load_gather(cnt_v, [(lane_iota + sh) & (LANES - 1)])
        return x

    def fixup(a, r0):
        buf = buf_v[a]
        fid = fid_v[a]
        acc = fid[pl.ds(0, LANES)]
        for g in range(1, CHUNK // LANES):
            acc = acc | fid[pl.ds(g * LANES, LANES)]

        @pl.when(or_tree(acc)[0] != 0)
        def _dirty_chunk():
            def group_body(g, _):
                fid16 = fid[pl.ds(g * LANES, LANES)]
                mask = fid16 != 0

                @pl.when(or_tree(fid16)[0] != 0)
                def _group():
                    row16 = g * LANES + lane_iota
                    pos16 = (r0 + row16) % L
                    for c0 in range(buf.shape[-1]):
                        col = jnp.full((LANES,), c0, jnp.int32)
                        vals = (plsc.load_gather(fair_v, [fid16, col])
                                + plsc.load_gather(pos_v, [pos16, col]))
                        plsc.store_scatter(buf, [row16, col], vals,
                                           mask=mask)

                return 0

            lax.fori_loop(0, CHUNK // LANES, group_body, 0)

    issue_in(0, 0)
    issue_ids(0, 0)
    wait_ids(0)
    issue_gather(0)

    def pair_body(p, _):
        for par in range(2):
            c = p * 2 + par
            a, b = par, 1 - par

            @pl.when(c + 1 < n_chunks)
            def _prefetch():
                @pl.when(c > 0)
                def _():
                    wait_out(b)
                issue_in(c + 1, b)
                issue_ids(c + 1, b)

            wait_in(a)
            wait_gather(a)
            fixup(a, r0_of(c))

            @pl.when(c + 1 < n_chunks)
            def _next_gather():
                wait_ids(b)
                issue_gather(b)

            issue_out(c, a)
        return 0

    lax.fori_loop(0, n_chunks // 2, pair_body, 0)

    wait_out(0)
    wait_out(1)


def kernel(input_ids, unfair_embeds, fair_table, pos_table, token_map):
    B, L = input_ids.shape
    D = unfair_embeds.shape[-1]
    N = B * L
    assert N % (NW * CHUNK) == 0 and (N // (NW * CHUNK)) % 2 == 0 and D == 128

    ids_flat = input_ids.reshape(N)
    unfair = unfair_embeds.reshape(N, D)
    pos_sl = pos_table[:L]

    mesh = plsc.VectorSubcoreMesh(core_axis_name="c", subcore_axis_name="s",
                                  num_cores=NC, num_subcores=NS)
    kfn = pl.kernel(
        functools.partial(_body, L),
        out_type=jax.ShapeDtypeStruct((N, D), jnp.float32),
        mesh=mesh,
        scratch_types=[
            pltpu.VMEM((CHUNK,), jnp.int32),
            pltpu.VMEM((CHUNK,), jnp.int32),
            pltpu.VMEM((CHUNK,), jnp.int32),
            pltpu.VMEM((CHUNK,), jnp.int32),
            pltpu.VMEM((CHUNK, D), jnp.float32),
            pltpu.VMEM((CHUNK, D), jnp.float32),
            pltpu.VMEM((fair_table.shape[0], D), jnp.float32),
            pltpu.VMEM((L, D), jnp.float32),
            pltpu.VMEM((128,), jnp.int32),
        ] + [pltpu.SemaphoreType.DMA] * 8,
        compiler_params=pltpu.CompilerParams(needs_layout_passes=False),
    )
    out = kfn(ids_flat, unfair, fair_table, pos_sl, token_map)
    return out.reshape(B, L, D)
